# Initial kernel scaffold; baseline (speedup 1.0000x reference)
import jax
import jax.numpy as jnp
from jax.experimental import pallas as pl


def kernel(input, W):
    raise NotImplementedError("write your pallas kernel here")


# speedup vs baseline: 2.7499x; 2.7499x over previous
import functools

import jax
import jax.numpy as jnp
from jax import lax
from jax.experimental import pallas as pl
from jax.experimental.pallas import tpu as pltpu
from jax.experimental.pallas import tpu_sc as plsc

VOCAB = 1000
EMB = 128
BATCH = 4096
HIST = 50

B = BATCH * HIST
NC = 2
NS = 16
NW = NC * NS
BPW = B // NW
CH = 128
NCH = BPW // CH


def _embed_flat(idx3, W):
    mesh = plsc.VectorSubcoreMesh(core_axis_name="c", subcore_axis_name="s")

    @functools.partial(
        pl.kernel,
        mesh=mesh,
        out_type=jax.ShapeDtypeStruct((B, EMB), jnp.float32),
        scratch_types=[
            pltpu.VMEM((NCH, CH), jnp.int32),
            pltpu.VMEM((CH, EMB), jnp.float32),
            pltpu.SemaphoreType.DMA,
        ],
    )
    def k(table_hbm, idx_hbm, out_hbm, idx_v, buf, sem):
        cid = lax.axis_index("c")
        sid = lax.axis_index("s")
        wid = sid * NC + cid
        base = wid * BPW

        pltpu.sync_copy(idx_hbm.at[wid], idx_v)

        def body(j, carry):
            pltpu.async_copy(table_hbm.at[idx_v.at[j]], buf, sem).wait()
            pltpu.sync_copy(buf, out_hbm.at[pl.ds(base + j * CH, CH)])
            return carry

        lax.fori_loop(0, NCH, body, 0)

    return k(W, idx3)


def kernel(input, W):
    idx3 = input.reshape(NW, NCH, CH)
    out = _embed_flat(idx3, W)
    return out.reshape(BATCH, HIST, EMB)


# speedup vs baseline: 2.8963x; 1.0532x over previous
import functools

import jax
import jax.numpy as jnp
from jax import lax
from jax.experimental import pallas as pl
from jax.experimental.pallas import tpu as pltpu
from jax.experimental.pallas import tpu_sc as plsc

VOCAB = 1000
EMB = 128
BATCH = 4096
HIST = 50

B = BATCH * HIST
NC = 2
NS = 16
NW = NC * NS
BPW = B // NW
CH = 128
NCH = BPW // CH
NB = 5
NP = NCH // NB


def _embed_flat(idx3, W):
    mesh = plsc.VectorSubcoreMesh(core_axis_name="c", subcore_axis_name="s")

    @functools.partial(
        pl.kernel,
        mesh=mesh,
        out_type=jax.ShapeDtypeStruct((B, EMB), jnp.float32),
        scratch_types=[
            pltpu.VMEM((NCH, CH), jnp.int32),
            pltpu.VMEM((NB, CH, EMB), jnp.float32),
            pltpu.SemaphoreType.DMA((NB,)),
            pltpu.SemaphoreType.DMA((NB,)),
        ],
    )
    def k(table_hbm, idx_hbm, out_hbm, idx_v, bufs, gsem, ssem):
        cid = lax.axis_index("c")
        sid = lax.axis_index("s")
        wid = sid * NC + cid
        base = wid * BPW

        pltpu.sync_copy(idx_hbm.at[wid], idx_v)

        def gather(j, b):
            pltpu.async_copy(table_hbm.at[idx_v.at[j]], bufs.at[b], gsem.at[b])

        def store(j, b):
            dst = out_hbm.at[pl.ds(base + j * CH, CH)]
            pltpu.async_copy(bufs.at[b], dst, ssem.at[b])
            return dst

        for b in range(NB):
            gather(b, b)

        def body(p, carry):
            for b in range(NB):
                j = p * NB + b
                pltpu.make_async_copy(
                    table_hbm.at[idx_v.at[j]], bufs.at[b], gsem.at[b]
                ).wait()
                dst = store(j, b)
                pltpu.make_async_copy(bufs.at[b], dst, ssem.at[b]).wait()
                gather(j + NB, b)
            return carry

        lax.fori_loop(0, NP - 1, body, 0)

        for b in range(NB):
            j = (NP - 1) * NB + b
            pltpu.make_async_copy(
                table_hbm.at[idx_v.at[j]], bufs.at[b], gsem.at[b]
            ).wait()
            dst = store(j, b)
            pltpu.make_async_copy(bufs.at[b], dst, ssem.at[b]).wait()

    return k(W, idx3)


def kernel(input, W):
    idx3 = input.reshape(NW, NCH, CH)
    out = _embed_flat(idx3, W)
    return out.reshape(BATCH, HIST, EMB)


# speedup vs baseline: 4.7281x; 1.6325x over previous
import functools

import jax
import jax.numpy as jnp
from jax import lax
from jax.experimental import pallas as pl
from jax.experimental.pallas import tpu as pltpu
from jax.experimental.pallas import tpu_sc as plsc

VOCAB = 1000
EMB = 128
BATCH = 4096
HIST = 50

NC = 2
NS = 16
NW = NC * NS
BPW = BATCH // NW
NB = 4
NP = BPW // NB


def _embed(idx, W):
    mesh = plsc.VectorSubcoreMesh(core_axis_name="c", subcore_axis_name="s")

    @functools.partial(
        pl.kernel,
        mesh=mesh,
        out_type=jax.ShapeDtypeStruct((BATCH, HIST, EMB), jnp.float32),
        scratch_types=[
            pltpu.VMEM((BPW, HIST), jnp.int32),
            pltpu.VMEM((NB, HIST, EMB), jnp.float32),
            pltpu.SemaphoreType.DMA((NB,)),
            pltpu.SemaphoreType.DMA((NB,)),
        ],
    )
    def k(table_hbm, idx_hbm, out_hbm, idx_v, bufs, gsem, ssem):
        cid = lax.axis_index("c")
        sid = lax.axis_index("s")
        wid = sid * NC + cid
        base = wid * BPW

        pltpu.sync_copy(idx_hbm.at[pl.ds(base, BPW)], idx_v)

        def gather(j, b):
            pltpu.async_copy(table_hbm.at[idx_v.at[j]], bufs.at[b], gsem.at[b])

        def store(j, b):
            dst = out_hbm.at[base + j]
            pltpu.async_copy(bufs.at[b], dst, ssem.at[b])
            return dst

        for b in range(NB):
            gather(b, b)

        def body(p, carry):
            for b in range(NB):
                j = p * NB + b
                pltpu.make_async_copy(
                    table_hbm.at[idx_v.at[j]], bufs.at[b], gsem.at[b]
                ).wait()
                dst = store(j, b)
                pltpu.make_async_copy(bufs.at[b], dst, ssem.at[b]).wait()
                gather(j + NB, b)
            return carry

        lax.fori_loop(0, NP - 1, body, 0)

        for b in range(NB):
            j = (NP - 1) * NB + b
            pltpu.make_async_copy(
                table_hbm.at[idx_v.at[j]], bufs.at[b], gsem.at[b]
            ).wait()
            dst = store(j, b)
            pltpu.make_async_copy(bufs.at[b], dst, ssem.at[b]).wait()

    return k(W, idx)


def kernel(input, W):
    return _embed(input, W)


# speedup vs baseline: 4.7365x; 1.0018x over previous
import functools

import jax
import jax.numpy as jnp
from jax import lax
from jax.experimental import pallas as pl
from jax.experimental.pallas import tpu as pltpu
from jax.experimental.pallas import tpu_sc as plsc

VOCAB = 1000
EMB = 128
BATCH = 4096
HIST = 50

NC = 2
NS = 16
NW = NC * NS
BPW = BATCH // NW
NB = 4
NP = BPW // NB


def _embed(idx, W):
    mesh = plsc.VectorSubcoreMesh(core_axis_name="c", subcore_axis_name="s")

    @functools.partial(
        pl.kernel,
        mesh=mesh,
        compiler_params=pltpu.CompilerParams(use_tc_tiling_on_sc=True),
        out_type=jax.ShapeDtypeStruct((BATCH, HIST, EMB), jnp.float32),
        scratch_types=[
            pltpu.VMEM((BPW, HIST), jnp.int32),
            pltpu.VMEM((NB, HIST, EMB), jnp.float32),
            pltpu.SemaphoreType.DMA((NB,)),
            pltpu.SemaphoreType.DMA((NB,)),
        ],
    )
    def k(table_hbm, idx_hbm, out_hbm, idx_v, bufs, gsem, ssem):
        cid = lax.axis_index("c")
        sid = lax.axis_index("s")
        wid = sid * NC + cid
        base = wid * BPW

        pltpu.sync_copy(idx_hbm.at[pl.ds(base, BPW)], idx_v)

        def gather(j, b):
            pltpu.async_copy(table_hbm.at[idx_v.at[j]], bufs.at[b], gsem.at[b])

        def store(j, b):
            dst = out_hbm.at[base + j]
            pltpu.async_copy(bufs.at[b], dst, ssem.at[b])
            return dst

        for b in range(NB):
            gather(b, b)

        def body(p, carry):
            for b in range(NB):
                j = p * NB + b
                pltpu.make_async_copy(
                    table_hbm.at[idx_v.at[j]], bufs.at[b], gsem.at[b]
                ).wait()
                dst = store(j, b)
                pltpu.make_async_copy(bufs.at[b], dst, ssem.at[b]).wait()
                gather(j + NB, b)
            return carry

        lax.fori_loop(0, NP - 1, body, 0)

        for b in range(NB):
            j = (NP - 1) * NB + b
            pltpu.make_async_copy(
                table_hbm.at[idx_v.at[j]], bufs.at[b], gsem.at[b]
            ).wait()
            dst = store(j, b)
            pltpu.make_async_copy(bufs.at[b], dst, ssem.at[b]).wait()

    return k(W, idx)


def kernel(input, W):
    return _embed(input, W)


# speedup vs baseline: 7.3127x; 1.5439x over previous
import functools

import jax
import jax.numpy as jnp
from jax import lax
from jax.experimental import pallas as pl
from jax.experimental.pallas import tpu as pltpu
from jax.experimental.pallas import tpu_sc as plsc

VOCAB = 1000
EMB = 128
BATCH = 4096
HIST = 50

NC = 2
NS = 16
NW = NC * NS
BPW = BATCH // NW
NB = 4
NP = BPW // NB


def _embed(idx, W):
    mesh = plsc.VectorSubcoreMesh(core_axis_name="c", subcore_axis_name="s")

    @functools.partial(
        pl.kernel,
        mesh=mesh,
        out_type=jax.ShapeDtypeStruct((BATCH, HIST, EMB), jnp.float32),
        scratch_types=[
            pltpu.VMEM((BPW, HIST), jnp.int32),
            pltpu.VMEM((NB, HIST, EMB), jnp.float32),
            pltpu.VMEM_SHARED((VOCAB, EMB), jnp.float32),
            pltpu.SemaphoreType.DMA((NB,)),
            pltpu.SemaphoreType.DMA((NB,)),
        ],
    )
    def k(table_hbm, idx_hbm, out_hbm, idx_v, bufs, tab_sh, gsem, ssem):
        cid = lax.axis_index("c")
        sid = lax.axis_index("s")
        wid = sid * NC + cid
        base = wid * BPW

        @pl.when(sid == 0)
        def _():
            pltpu.sync_copy(table_hbm, tab_sh)

        pltpu.sync_copy(idx_hbm.at[pl.ds(base, BPW)], idx_v)
        plsc.subcore_barrier()

        def gather(j, b):
            pltpu.async_copy(tab_sh.at[idx_v.at[j]], bufs.at[b], gsem.at[b])

        def store(j, b):
            dst = out_hbm.at[base + j]
            pltpu.async_copy(bufs.at[b], dst, ssem.at[b])
            return dst

        for b in range(NB):
            gather(b, b)

        def body(p, carry):
            for b in range(NB):
                j = p * NB + b
                pltpu.make_async_copy(
                    tab_sh.at[idx_v.at[j]], bufs.at[b], gsem.at[b]
                ).wait()
                dst = store(j, b)
                pltpu.make_async_copy(bufs.at[b], dst, ssem.at[b]).wait()
                gather(j + NB, b)
            return carry

        lax.fori_loop(0, NP - 1, body, 0)

        for b in range(NB):
            j = (NP - 1) * NB + b
            pltpu.make_async_copy(
                tab_sh.at[idx_v.at[j]], bufs.at[b], gsem.at[b]
            ).wait()
            dst = store(j, b)
            pltpu.make_async_copy(bufs.at[b], dst, ssem.at[b]).wait()

    return k(W, idx)


def kernel(input, W):
    return _embed(input, W)
